# baseline (speedup 1.0000x reference)
import functools

import jax
import jax.numpy as jnp
from jax import lax
from jax.experimental import pallas as pl
from jax.experimental.pallas import tpu as pltpu
from jax.experimental.pallas import tpu_sc as plsc

N = 10000
E = 320000
D_IN = 128
H1 = 64

NC = 2
NS = 16
NW = NC * NS
NPAD = 10240
ROWS_PER_TILE = NPAD // NS
EPT = E // NW
CHUNK = 128
NBUF = 4
NCHUNK = 80
EPT_PAD = NCHUNK * CHUNK
PADROW = NPAD - 1

_mesh = plsc.VectorSubcoreMesh(core_axis_name="c", subcore_axis_name="s")
_sc_params = pltpu.CompilerParams(
    needs_layout_passes=False, use_tc_tiling_on_sc=False)


@functools.partial(
    pl.kernel,
    out_type=jax.ShapeDtypeStruct((NW, NPAD), jnp.float32),
    mesh=_mesh,
    compiler_params=_sc_params,
    scratch_types=[
        pltpu.VMEM((EPT_PAD,), jnp.int32),
        pltpu.VMEM((NPAD,), jnp.float32),
    ],
)
def _deg_kernel(dst_hbm, degp_hbm, idx_v, hist_v):
    c = lax.axis_index("c")
    s = lax.axis_index("s")
    wid = s * NC + c
    pltpu.sync_copy(dst_hbm.at[wid], idx_v)

    def zero_body(i, _):
        hist_v[pl.ds(i * 16, 16)] = jnp.zeros((16,), jnp.float32)
        return 0

    lax.fori_loop(0, NPAD // 16, zero_body, 0)
    ones = jnp.ones((16,), jnp.float32)

    def body(g, _):
        idx = idx_v[pl.ds(g * 16, 16)]
        plsc.addupdate_scatter(hist_v, [idx], ones)
        return 0

    lax.fori_loop(0, EPT_PAD // 16, body, 0)
    pltpu.sync_copy(hist_v, degp_hbm.at[wid])


@functools.partial(
    pl.kernel,
    out_type=jax.ShapeDtypeStruct((NC, NPAD, H1), jnp.float32),
    mesh=_mesh,
    compiler_params=_sc_params,
    scratch_types=[
        pltpu.VMEM((NCHUNK, CHUNK), jnp.int32),
        pltpu.VMEM((NCHUNK, CHUNK), jnp.int32),
    ] + [pltpu.VMEM((CHUNK, H1), jnp.float32) for _ in range(NBUF)]
      + [pltpu.SemaphoreType.DMA for _ in range(NBUF)]
      + [pltpu.VMEM_SHARED((NPAD, H1), jnp.float32)],
)
def _agg_kernel(u_hbm, uh_hbm, src_hbm, dst_hbm, out_hbm, src_v, dst_v, *scr):
    rows = scr[0:NBUF]
    gsem = scr[NBUF:2 * NBUF]
    acc = scr[2 * NBUF]
    c = lax.axis_index("c")
    s = lax.axis_index("s")
    wid = s * NC + c
    pltpu.sync_copy(src_hbm.at[wid], src_v)
    pltpu.sync_copy(dst_hbm.at[wid], dst_v)
    base = s * ROWS_PER_TILE
    pltpu.sync_copy(uh_hbm.at[pl.ds(base, ROWS_PER_TILE)],
                    acc.at[pl.ds(base, ROWS_PER_TILE)])
    plsc.subcore_barrier()

    for b in range(NBUF):
        pltpu.async_copy(u_hbm.at[src_v.at[b]], rows[b], gsem[b])

    def outer(t, _):
        cbase = t * NBUF
        for b in range(NBUF):
            j = cbase + b
            pltpu.make_async_copy(u_hbm.at[src_v.at[j]], rows[b], gsem[b]).wait()
            pltpu.sync_copy(rows[b], acc.at[dst_v.at[j]], add=True)

            @pl.when(j + NBUF < NCHUNK)
            def _():
                pltpu.async_copy(u_hbm.at[src_v.at[j + NBUF]], rows[b], gsem[b])
        return 0

    lax.fori_loop(0, NCHUNK // NBUF, outer, 0)
    plsc.subcore_barrier()
    pltpu.sync_copy(acc.at[pl.ds(base, ROWS_PER_TILE)],
                    out_hbm.at[c, pl.ds(base, ROWS_PER_TILE)])


BLK = 512
GRID = NPAD // BLK


def _k2_body(x_ref, w0_ref, degp_ref, u0_ref, u0h_ref, dinv_ref):
    deg = jnp.sum(degp_ref[...], axis=0) + 1.0
    dinv = lax.rsqrt(deg)
    y0 = jnp.dot(x_ref[...], w0_ref[...], preferred_element_type=jnp.float32)
    u0 = y0 * dinv[:, None]
    u0_ref[...] = u0
    u0h_ref[...] = 0.5 * u0
    dinv_ref[...] = dinv


def _k4_body(p_ref, dinv_ref, u1_ref, u1h_ref):
    p = p_ref[...]
    d = dinv_ref[...]
    u1 = (d * d)[:, None] * (p[0] + p[1])
    u1_ref[...] = u1
    u1h_ref[...] = 0.5 * u1


def _sigmoid(v):
    return 1.0 / (1.0 + jnp.exp(-v))


def _softplus(v):
    return jnp.maximum(v, 0.0) + jnp.log(1.0 + jnp.exp(-jnp.abs(v)))


def _k5_body(q_ref, dinv_ref, wcat_ref, fblk_ref, zm_ref, zs_ref, zp_ref, za_ref):
    q = q_ref[...]
    g = dinv_ref[...][:, None] * (q[0] + q[1])
    h = _sigmoid(jnp.dot(g, wcat_ref[...], preferred_element_type=jnp.float32))
    z = jnp.dot(h, fblk_ref[...], preferred_element_type=jnp.float32)
    zm_ref[...] = z[:, 0:32]
    zs_ref[...] = z[:, 32:64]
    zp_ref[...] = z[:, 64:96]
    za_ref[...] = _softplus(z[:, 96:128])


def kernel(x, edge_index, W0, Wm, Ws, Wp, Wa, Fm, Fs, Fp, Fa):
    src = edge_index[0]
    dst = edge_index[1]
    src_t = jnp.pad(src.reshape(NW, EPT), ((0, 0), (0, EPT_PAD - EPT)),
                    constant_values=PADROW).reshape(NW, NCHUNK, CHUNK)
    dst_t = jnp.pad(dst.reshape(NW, EPT), ((0, 0), (0, EPT_PAD - EPT)),
                    constant_values=PADROW).reshape(NW, NCHUNK, CHUNK)
    dst_f = dst_t.reshape(NW, EPT_PAD)
    x_pad = jnp.pad(x, ((0, NPAD - N), (0, 0)))
    wcat = jnp.concatenate([Wm, Ws, Wp, Wa], axis=1)
    fblk = jax.scipy.linalg.block_diag(Fm, Fs, Fp, Fa)

    degp = _deg_kernel(dst_f)

    u0, u0h, dinv = pl.pallas_call(
        _k2_body,
        grid=(GRID,),
        in_specs=[
            pl.BlockSpec((BLK, D_IN), lambda i: (i, 0)),
            pl.BlockSpec((D_IN, H1), lambda i: (0, 0)),
            pl.BlockSpec((NW, BLK), lambda i: (0, i)),
        ],
        out_specs=[
            pl.BlockSpec((BLK, H1), lambda i: (i, 0)),
            pl.BlockSpec((BLK, H1), lambda i: (i, 0)),
            pl.BlockSpec((BLK,), lambda i: (i,)),
        ],
        out_shape=[
            jax.ShapeDtypeStruct((NPAD, H1), jnp.float32),
            jax.ShapeDtypeStruct((NPAD, H1), jnp.float32),
            jax.ShapeDtypeStruct((NPAD,), jnp.float32),
        ],
    )(x_pad, W0, degp)

    p = _agg_kernel(u0, u0h, src_t, dst_t)

    u1, u1h = pl.pallas_call(
        _k4_body,
        grid=(GRID,),
        in_specs=[
            pl.BlockSpec((NC, BLK, H1), lambda i: (0, i, 0)),
            pl.BlockSpec((BLK,), lambda i: (i,)),
        ],
        out_specs=[
            pl.BlockSpec((BLK, H1), lambda i: (i, 0)),
            pl.BlockSpec((BLK, H1), lambda i: (i, 0)),
        ],
        out_shape=[
            jax.ShapeDtypeStruct((NPAD, H1), jnp.float32),
            jax.ShapeDtypeStruct((NPAD, H1), jnp.float32),
        ],
    )(p, dinv)

    q = _agg_kernel(u1, u1h, src_t, dst_t)

    zm, zs, zp, za = pl.pallas_call(
        _k5_body,
        grid=(GRID,),
        in_specs=[
            pl.BlockSpec((NC, BLK, H1), lambda i: (0, i, 0)),
            pl.BlockSpec((BLK,), lambda i: (i,)),
            pl.BlockSpec((H1, 128), lambda i: (0, 0)),
            pl.BlockSpec((128, 128), lambda i: (0, 0)),
        ],
        out_specs=[pl.BlockSpec((BLK, 32), lambda i: (i, 0))] * 4,
        out_shape=[jax.ShapeDtypeStruct((NPAD, 32), jnp.float32)] * 4,
    )(q, dinv, wcat, fblk)

    return jnp.stack([zm, zs, zp, za])[:, :N]


# speedup vs baseline: 1.7244x; 1.7244x over previous
import functools

import jax
import jax.numpy as jnp
from jax import lax
from jax.experimental import pallas as pl
from jax.experimental.pallas import tpu as pltpu
from jax.experimental.pallas import tpu_sc as plsc

N = 10000
E = 320000
D_IN = 128
H1 = 64

NC = 2
NS = 16
NW = NC * NS
NPAD = 10240
ROWS_PER_TILE = NPAD // NS
EPT = E // NW
CHUNK = 128
NBUF = 2
NCHUNK = 80
EPT_PAD = NCHUNK * CHUNK
PADROW = NPAD - 1

_mesh = plsc.VectorSubcoreMesh(core_axis_name="c", subcore_axis_name="s")
_sc_params = pltpu.CompilerParams(
    needs_layout_passes=False, use_tc_tiling_on_sc=False)


@functools.partial(
    pl.kernel,
    out_type=jax.ShapeDtypeStruct((NW, NPAD), jnp.float32),
    mesh=_mesh,
    compiler_params=_sc_params,
    scratch_types=[
        pltpu.VMEM((EPT_PAD,), jnp.int32),
        pltpu.VMEM((NPAD,), jnp.float32),
    ],
)
def _deg_kernel(dst_hbm, degp_hbm, idx_v, hist_v):
    c = lax.axis_index("c")
    s = lax.axis_index("s")
    wid = s * NC + c
    pltpu.sync_copy(dst_hbm.at[wid], idx_v)

    def zero_body(i, _):
        hist_v[pl.ds(i * 16, 16)] = jnp.zeros((16,), jnp.float32)
        return 0

    lax.fori_loop(0, NPAD // 16, zero_body, 0)
    ones = jnp.ones((16,), jnp.float32)

    def body(g, _):
        idx = idx_v[pl.ds(g * 16, 16)]
        plsc.addupdate_scatter(hist_v, [idx], ones)
        return 0

    lax.fori_loop(0, EPT_PAD // 16, body, 0)
    pltpu.sync_copy(hist_v, degp_hbm.at[wid])


@functools.partial(
    pl.kernel,
    out_type=jax.ShapeDtypeStruct((NC, NPAD, H1), jnp.float32),
    mesh=_mesh,
    compiler_params=_sc_params,
    scratch_types=[
        pltpu.VMEM((NCHUNK, CHUNK), jnp.int32),
        pltpu.VMEM((NCHUNK, CHUNK), jnp.int32),
    ] + [pltpu.VMEM((CHUNK, H1), jnp.float32) for _ in range(NBUF)]
      + [pltpu.SemaphoreType.DMA for _ in range(NBUF)]
      + [pltpu.VMEM_SHARED((NPAD, H1), jnp.float32),
         pltpu.VMEM_SHARED((NPAD, H1), jnp.float32)],
)
def _agg_kernel(u_hbm, uh_hbm, src_hbm, dst_hbm, out_hbm, src_v, dst_v, *scr):
    rows = scr[0:NBUF]
    gsem = scr[NBUF:2 * NBUF]
    acc = scr[2 * NBUF]
    u_s = scr[2 * NBUF + 1]
    c = lax.axis_index("c")
    s = lax.axis_index("s")
    wid = s * NC + c
    pltpu.sync_copy(src_hbm.at[wid], src_v)
    pltpu.sync_copy(dst_hbm.at[wid], dst_v)
    base = s * ROWS_PER_TILE
    pltpu.sync_copy(u_hbm.at[pl.ds(base, ROWS_PER_TILE)],
                    u_s.at[pl.ds(base, ROWS_PER_TILE)])
    pltpu.sync_copy(uh_hbm.at[pl.ds(base, ROWS_PER_TILE)],
                    acc.at[pl.ds(base, ROWS_PER_TILE)])
    plsc.subcore_barrier()

    for b in range(NBUF):
        pltpu.async_copy(u_s.at[src_v.at[b]], rows[b], gsem[b])

    def outer(t, _):
        cbase = t * NBUF
        for b in range(NBUF):
            j = cbase + b
            pltpu.make_async_copy(u_s.at[src_v.at[j]], rows[b], gsem[b]).wait()
            pltpu.sync_copy(rows[b], acc.at[dst_v.at[j]], add=True)

            @pl.when(j + NBUF < NCHUNK)
            def _():
                pltpu.async_copy(u_s.at[src_v.at[j + NBUF]], rows[b], gsem[b])
        return 0

    lax.fori_loop(0, NCHUNK // NBUF, outer, 0)
    plsc.subcore_barrier()
    pltpu.sync_copy(acc.at[pl.ds(base, ROWS_PER_TILE)],
                    out_hbm.at[c, pl.ds(base, ROWS_PER_TILE)])


BLK = 512
GRID = NPAD // BLK


def _k2_body(x_ref, w0_ref, degp_ref, u0_ref, u0h_ref, dinv_ref):
    deg = jnp.sum(degp_ref[...], axis=0) + 1.0
    dinv = lax.rsqrt(deg)
    y0 = jnp.dot(x_ref[...], w0_ref[...], preferred_element_type=jnp.float32)
    u0 = y0 * dinv[:, None]
    u0_ref[...] = u0
    u0h_ref[...] = 0.5 * u0
    dinv_ref[...] = dinv


def _k4_body(p_ref, dinv_ref, u1_ref, u1h_ref):
    p = p_ref[...]
    d = dinv_ref[...]
    u1 = (d * d)[:, None] * (p[0] + p[1])
    u1_ref[...] = u1
    u1h_ref[...] = 0.5 * u1


def _sigmoid(v):
    return 1.0 / (1.0 + jnp.exp(-v))


def _softplus(v):
    return jnp.maximum(v, 0.0) + jnp.log(1.0 + jnp.exp(-jnp.abs(v)))


def _k5_body(q_ref, dinv_ref, wcat_ref, fblk_ref, zm_ref, zs_ref, zp_ref, za_ref):
    q = q_ref[...]
    g = dinv_ref[...][:, None] * (q[0] + q[1])
    h = _sigmoid(jnp.dot(g, wcat_ref[...], preferred_element_type=jnp.float32))
    z = jnp.dot(h, fblk_ref[...], preferred_element_type=jnp.float32)
    zm_ref[...] = z[:, 0:32]
    zs_ref[...] = z[:, 32:64]
    zp_ref[...] = z[:, 64:96]
    za_ref[...] = _softplus(z[:, 96:128])


def kernel(x, edge_index, W0, Wm, Ws, Wp, Wa, Fm, Fs, Fp, Fa):
    src = edge_index[0]
    dst = edge_index[1]
    src_t = jnp.pad(src.reshape(NW, EPT), ((0, 0), (0, EPT_PAD - EPT)),
                    constant_values=PADROW).reshape(NW, NCHUNK, CHUNK)
    dst_t = jnp.pad(dst.reshape(NW, EPT), ((0, 0), (0, EPT_PAD - EPT)),
                    constant_values=PADROW).reshape(NW, NCHUNK, CHUNK)
    dst_f = dst_t.reshape(NW, EPT_PAD)
    x_pad = jnp.pad(x, ((0, NPAD - N), (0, 0)))
    wcat = jnp.concatenate([Wm, Ws, Wp, Wa], axis=1)
    fblk = jax.scipy.linalg.block_diag(Fm, Fs, Fp, Fa)

    degp = _deg_kernel(dst_f)

    u0, u0h, dinv = pl.pallas_call(
        _k2_body,
        grid=(GRID,),
        in_specs=[
            pl.BlockSpec((BLK, D_IN), lambda i: (i, 0)),
            pl.BlockSpec((D_IN, H1), lambda i: (0, 0)),
            pl.BlockSpec((NW, BLK), lambda i: (0, i)),
        ],
        out_specs=[
            pl.BlockSpec((BLK, H1), lambda i: (i, 0)),
            pl.BlockSpec((BLK, H1), lambda i: (i, 0)),
            pl.BlockSpec((BLK,), lambda i: (i,)),
        ],
        out_shape=[
            jax.ShapeDtypeStruct((NPAD, H1), jnp.float32),
            jax.ShapeDtypeStruct((NPAD, H1), jnp.float32),
            jax.ShapeDtypeStruct((NPAD,), jnp.float32),
        ],
    )(x_pad, W0, degp)

    p = _agg_kernel(u0, u0h, src_t, dst_t)

    u1, u1h = pl.pallas_call(
        _k4_body,
        grid=(GRID,),
        in_specs=[
            pl.BlockSpec((NC, BLK, H1), lambda i: (0, i, 0)),
            pl.BlockSpec((BLK,), lambda i: (i,)),
        ],
        out_specs=[
            pl.BlockSpec((BLK, H1), lambda i: (i, 0)),
            pl.BlockSpec((BLK, H1), lambda i: (i, 0)),
        ],
        out_shape=[
            jax.ShapeDtypeStruct((NPAD, H1), jnp.float32),
            jax.ShapeDtypeStruct((NPAD, H1), jnp.float32),
        ],
    )(p, dinv)

    q = _agg_kernel(u1, u1h, src_t, dst_t)

    zm, zs, zp, za = pl.pallas_call(
        _k5_body,
        grid=(GRID,),
        in_specs=[
            pl.BlockSpec((NC, BLK, H1), lambda i: (0, i, 0)),
            pl.BlockSpec((BLK,), lambda i: (i,)),
            pl.BlockSpec((H1, 128), lambda i: (0, 0)),
            pl.BlockSpec((128, 128), lambda i: (0, 0)),
        ],
        out_specs=[pl.BlockSpec((BLK, 32), lambda i: (i, 0))] * 4,
        out_shape=[jax.ShapeDtypeStruct((NPAD, 32), jnp.float32)] * 4,
    )(q, dinv, wcat, fblk)

    return jnp.stack([zm, zs, zp, za])[:, :N]
